# speedup vs baseline: 1.9341x; 1.9341x over previous
import numpy as np
import jax
import jax.numpy as jnp
from jax.experimental import pallas as pl

VOCAB = 100000
MASK_ID = 4

_ROT0 = (13, 15, 26, 6)
_ROT1 = (17, 29, 16, 24)


def _np_rotl(x, d):
    return ((x << np.uint32(d)) | (x >> np.uint32(32 - d))).astype(np.uint32)


def _np_threefry(k1, k2, x1, x2):
    k1 = np.uint32(k1); k2 = np.uint32(k2)
    x1 = np.asarray(x1, np.uint32).copy(); x2 = np.asarray(x2, np.uint32).copy()
    ks = (k1, k2, np.uint32(np.uint32(0x1BD11BDA) ^ k1 ^ k2))
    x1 = (x1 + ks[0]).astype(np.uint32); x2 = (x2 + ks[1]).astype(np.uint32)
    def rounds(x1, x2, rots):
        for r in rots:
            x1 = (x1 + x2).astype(np.uint32)
            x2 = _np_rotl(x2, r)
            x2 = (x1 ^ x2).astype(np.uint32)
        return x1, x2
    inj = ((ks[1], ks[2], 1), (ks[2], ks[0], 2), (ks[0], ks[1], 3),
           (ks[1], ks[2], 4), (ks[2], ks[0], 5))
    rots = (_ROT0, _ROT1, _ROT0, _ROT1, _ROT0)
    for (a, b, c), rr in zip(inj, rots):
        x1, x2 = rounds(x1, x2, rr)
        x1 = (x1 + a).astype(np.uint32)
        x2 = (x2 + b + np.uint32(c)).astype(np.uint32)
    return x1, x2


def _derive_keys():
    cnt = np.arange(4, dtype=np.uint64)
    hi = (cnt >> np.uint64(32)).astype(np.uint32)
    lo = (cnt & np.uint64(0xFFFFFFFF)).astype(np.uint32)
    a, b = _np_threefry(0, 42, hi, lo)
    kb1 = (int(a[0]), int(b[0]))
    kb2 = (int(a[1]), int(b[1]))
    kb3 = (int(a[2]), int(b[2]))
    kr = (int(a[3]), int(b[3]))
    cnt = np.arange(2, dtype=np.uint64)
    hi = (cnt >> np.uint64(32)).astype(np.uint32)
    lo = (cnt & np.uint64(0xFFFFFFFF)).astype(np.uint32)
    a, b = _np_threefry(kr[0], kr[1], hi, lo)
    kr2 = (int(a[1]), int(b[1]))
    return kb1, kb2, kb3, kr2


_KB1, _KB2, _KB3, _KR2 = _derive_keys()

_T_MASK = int(np.ceil(np.float64(np.float32(0.15)) * 2**23))
_T_REPL = int(np.ceil(np.float64(np.float32(0.8)) * 2**23))
_T_RAND = int(np.ceil(np.float64(np.float32(0.5)) * 2**23))


def _rotl(x, d):
    return (x << np.uint32(d)) | (x >> np.uint32(32 - d))


def _tf_bits(key, lo):
    k1 = key[0] & 0xFFFFFFFF
    k2 = key[1] & 0xFFFFFFFF
    k3 = (0x1BD11BDA ^ k1 ^ k2) & 0xFFFFFFFF
    ks = (k1, k2, k3)
    x1 = jnp.full_like(lo, np.uint32(k1))
    x2 = lo + np.uint32(k2)
    rots = (_ROT0, _ROT1, _ROT0, _ROT1, _ROT0)
    inj = ((ks[1], (ks[2] + 1) & 0xFFFFFFFF),
           (ks[2], (ks[0] + 2) & 0xFFFFFFFF),
           (ks[0], (ks[1] + 3) & 0xFFFFFFFF),
           (ks[1], (ks[2] + 4) & 0xFFFFFFFF),
           (ks[2], (ks[0] + 5) & 0xFFFFFFFF))
    for rr, (a, b) in zip(rots, inj):
        for r in rr:
            x1 = x1 + x2
            x2 = _rotl(x2, r)
            x2 = x1 ^ x2
        x1 = x1 + np.uint32(a)
        x2 = x2 + np.uint32(b)
    return x1 ^ x2


def _collator_block(ids_ref, sp_ref, out_ref, lab_ref):
    i = pl.program_id(0)
    ids = ids_ref[...]
    sp = sp_ref[...]
    rows, cols = ids.shape
    r = jax.lax.broadcasted_iota(jnp.uint32, (rows, cols), 0)
    c = jax.lax.broadcasted_iota(jnp.uint32, (rows, cols), 1)
    n = i.astype(jnp.uint32) * np.uint32(rows * cols) + r * np.uint32(cols) + c

    b1 = _tf_bits(_KB1, n)
    masked = ((b1 >> np.uint32(9)) < np.uint32(_T_MASK)) & (~sp)
    lab_ref[...] = jnp.where(masked, ids, jnp.int32(-100))

    b2 = _tf_bits(_KB2, n)
    replaced = ((b2 >> np.uint32(9)) < np.uint32(_T_REPL)) & masked
    b3 = _tf_bits(_KB3, n)
    randm = ((b3 >> np.uint32(9)) < np.uint32(_T_RAND)) & masked & (~replaced)
    lowbits = _tf_bits(_KR2, n)
    rword = (lowbits % np.uint32(VOCAB)).astype(jnp.int32)

    out = jnp.where(replaced, jnp.int32(MASK_ID), ids)
    out_ref[...] = jnp.where(randm, rword, out)


def kernel(input_ids, special_tokens_mask):
    B, S = input_ids.shape
    rows = 16
    grid = (B // rows,)
    out_ids, labels = pl.pallas_call(
        _collator_block,
        grid=grid,
        in_specs=[
            pl.BlockSpec((rows, S), lambda i: (i, 0)),
            pl.BlockSpec((rows, S), lambda i: (i, 0)),
        ],
        out_specs=[
            pl.BlockSpec((rows, S), lambda i: (i, 0)),
            pl.BlockSpec((rows, S), lambda i: (i, 0)),
        ],
        out_shape=[
            jax.ShapeDtypeStruct((B, S), jnp.int32),
            jax.ShapeDtypeStruct((B, S), jnp.int32),
        ],
    )(input_ids, special_tokens_mask)
    return (out_ids, labels)


# speedup vs baseline: 1.9507x; 1.0085x over previous
import numpy as np
import jax
import jax.numpy as jnp
from jax.experimental import pallas as pl
from jax.experimental.pallas import tpu as pltpu

VOCAB = 100000
MASK_ID = 4

_ROT0 = (13, 15, 26, 6)
_ROT1 = (17, 29, 16, 24)


def _np_rotl(x, d):
    return ((x << np.uint32(d)) | (x >> np.uint32(32 - d))).astype(np.uint32)


def _np_threefry(k1, k2, x1, x2):
    k1 = np.uint32(k1); k2 = np.uint32(k2)
    x1 = np.asarray(x1, np.uint32).copy(); x2 = np.asarray(x2, np.uint32).copy()
    ks = (k1, k2, np.uint32(np.uint32(0x1BD11BDA) ^ k1 ^ k2))
    x1 = (x1 + ks[0]).astype(np.uint32); x2 = (x2 + ks[1]).astype(np.uint32)
    def rounds(x1, x2, rots):
        for r in rots:
            x1 = (x1 + x2).astype(np.uint32)
            x2 = _np_rotl(x2, r)
            x2 = (x1 ^ x2).astype(np.uint32)
        return x1, x2
    inj = ((ks[1], ks[2], 1), (ks[2], ks[0], 2), (ks[0], ks[1], 3),
           (ks[1], ks[2], 4), (ks[2], ks[0], 5))
    rots = (_ROT0, _ROT1, _ROT0, _ROT1, _ROT0)
    for (a, b, c), rr in zip(inj, rots):
        x1, x2 = rounds(x1, x2, rr)
        x1 = (x1 + a).astype(np.uint32)
        x2 = (x2 + b + np.uint32(c)).astype(np.uint32)
    return x1, x2


def _derive_keys():
    cnt = np.arange(4, dtype=np.uint64)
    hi = (cnt >> np.uint64(32)).astype(np.uint32)
    lo = (cnt & np.uint64(0xFFFFFFFF)).astype(np.uint32)
    a, b = _np_threefry(0, 42, hi, lo)
    kb1 = (int(a[0]), int(b[0]))
    kb2 = (int(a[1]), int(b[1]))
    kb3 = (int(a[2]), int(b[2]))
    kr = (int(a[3]), int(b[3]))
    cnt = np.arange(2, dtype=np.uint64)
    hi = (cnt >> np.uint64(32)).astype(np.uint32)
    lo = (cnt & np.uint64(0xFFFFFFFF)).astype(np.uint32)
    a, b = _np_threefry(kr[0], kr[1], hi, lo)
    kr2 = (int(a[1]), int(b[1]))
    return kb1, kb2, kb3, kr2


_KB1, _KB2, _KB3, _KR2 = _derive_keys()

_T_MASK = int(np.ceil(np.float64(np.float32(0.15)) * 2**23))
_T_REPL = int(np.ceil(np.float64(np.float32(0.8)) * 2**23))
_T_RAND = int(np.ceil(np.float64(np.float32(0.5)) * 2**23))


def _rotl(x, d):
    return (x << np.uint32(d)) | (x >> np.uint32(32 - d))


def _tf_bits(key, lo):
    k1 = key[0] & 0xFFFFFFFF
    k2 = key[1] & 0xFFFFFFFF
    k3 = (0x1BD11BDA ^ k1 ^ k2) & 0xFFFFFFFF
    ks = (k1, k2, k3)
    x1 = jnp.full_like(lo, np.uint32(k1))
    x2 = lo + np.uint32(k2)
    rots = (_ROT0, _ROT1, _ROT0, _ROT1, _ROT0)
    inj = ((ks[1], (ks[2] + 1) & 0xFFFFFFFF),
           (ks[2], (ks[0] + 2) & 0xFFFFFFFF),
           (ks[0], (ks[1] + 3) & 0xFFFFFFFF),
           (ks[1], (ks[2] + 4) & 0xFFFFFFFF),
           (ks[2], (ks[0] + 5) & 0xFFFFFFFF))
    for rr, (a, b) in zip(rots, inj):
        for r in rr:
            x1 = x1 + x2
            x2 = _rotl(x2, r)
            x2 = x1 ^ x2
        x1 = x1 + np.uint32(a)
        x2 = x2 + np.uint32(b)
    return x1 ^ x2


def _collator_block(ids_ref, sp_ref, out_ref, lab_ref):
    i = pl.program_id(0)
    ids = ids_ref[...]
    sp = sp_ref[...]
    rows, cols = ids.shape
    r = jax.lax.broadcasted_iota(jnp.uint32, (rows, cols), 0)
    c = jax.lax.broadcasted_iota(jnp.uint32, (rows, cols), 1)
    n = i.astype(jnp.uint32) * np.uint32(rows * cols) + r * np.uint32(cols) + c

    b1 = _tf_bits(_KB1, n)
    masked = (b1 < np.uint32(_T_MASK * 512)) & (~sp)
    lab_ref[...] = jnp.where(masked, ids, jnp.int32(-100))

    b2 = _tf_bits(_KB2, n)
    replaced = (b2 < np.uint32(_T_REPL * 512)) & masked
    b3 = _tf_bits(_KB3, n)
    randm = (b3 < np.uint32(_T_RAND * 512)) & masked & (~replaced)
    lowbits = _tf_bits(_KR2, n)
    rword = (lowbits % np.uint32(VOCAB)).astype(jnp.int32)

    out = jnp.where(replaced, jnp.int32(MASK_ID), ids)
    out_ref[...] = jnp.where(randm, rword, out)


def kernel(input_ids, special_tokens_mask):
    B, S = input_ids.shape
    rows = 16
    grid = (B // rows,)
    out_ids, labels = pl.pallas_call(
        _collator_block,
        grid=grid,
        in_specs=[
            pl.BlockSpec((rows, S), lambda i: (i, 0)),
            pl.BlockSpec((rows, S), lambda i: (i, 0)),
        ],
        out_specs=[
            pl.BlockSpec((rows, S), lambda i: (i, 0)),
            pl.BlockSpec((rows, S), lambda i: (i, 0)),
        ],
        out_shape=[
            jax.ShapeDtypeStruct((B, S), jnp.int32),
            jax.ShapeDtypeStruct((B, S), jnp.int32),
        ],
        compiler_params=pltpu.CompilerParams(
            dimension_semantics=("parallel",),
        ),
    )(input_ids, special_tokens_mask)
    return (out_ids, labels)
